# baseline (speedup 1.0000x reference)
import functools

import jax
import jax.numpy as jnp
from jax.experimental import pallas as pl

T = 32768
D = 768
E = 64
K = 8
BT = 512

NEG_INF = float("-inf")


def _gate_kernel(x_ref, wt_ref, idx_ref, w_ref):
    x = x_ref[...]
    wt = wt_ref[...]
    logits = jnp.dot(x, wt, preferred_element_type=jnp.float32)
    lanes = jax.lax.broadcasted_iota(jnp.int32, (BT, E), 1)

    vals = []
    idxs = []
    work = logits
    for _ in range(K):
        m = jnp.max(work, axis=1, keepdims=True)
        hit = work == m
        idx = jnp.min(jnp.where(hit, lanes, E), axis=1, keepdims=True)
        vals.append(m)
        idxs.append(idx)
        work = jnp.where(lanes == idx, NEG_INF, work)

    v = jnp.concatenate(vals, axis=1)
    e = jnp.exp(v - v[:, 0:1])
    w = e / jnp.sum(e, axis=1, keepdims=True)
    idx_ref[...] = jnp.concatenate(idxs, axis=1)
    w_ref[...] = w


@jax.jit
def kernel(hidden_states, wg_weight):
    wt = wg_weight.astype(jnp.float32).T
    x = hidden_states.astype(jnp.float32)
    grid = (T // BT,)
    idx, w = pl.pallas_call(
        _gate_kernel,
        grid=grid,
        in_specs=[
            pl.BlockSpec((BT, D), lambda i: (i, 0)),
            pl.BlockSpec((D, E), lambda i: (0, 0)),
        ],
        out_specs=[
            pl.BlockSpec((BT, K), lambda i: (i, 0)),
            pl.BlockSpec((BT, K), lambda i: (i, 0)),
        ],
        out_shape=[
            jax.ShapeDtypeStruct((T, K), jnp.int32),
            jax.ShapeDtypeStruct((T, K), jnp.float32),
        ],
    )(x, wt)
    return idx, w.astype(hidden_states.dtype)


# speedup vs baseline: 1.6903x; 1.6903x over previous
import jax
import jax.numpy as jnp
from jax.experimental import pallas as pl

T = 32768
D = 768
E = 64
K = 8
BT = 512
CH = 64

NEG_INF = float("-inf")


def _gate_kernel(x_ref, wt_ref, idx_ref, w_ref):
    wt = wt_ref[...]
    lanes_f = jax.lax.broadcasted_iota(jnp.int32, (CH, E), 1).astype(jnp.float32)
    for c in range(BT // CH):
        rows = pl.ds(c * CH, CH)
        x = x_ref[rows, :]
        logits = jnp.dot(x, wt, preferred_element_type=jnp.float32)

        vals = []
        work = logits
        for k in range(K):
            m = jnp.max(work, axis=1, keepdims=True)
            vals.append(m)
            if k + 1 < K:
                work = jnp.where(work == m, NEG_INF, work)

        for k in range(K):
            hit = logits == vals[k]
            idxf = jnp.min(jnp.where(hit, lanes_f, jnp.float32(E)), axis=1,
                           keepdims=True)
            idx_ref[rows, k:k + 1] = idxf.astype(jnp.int32)

        es = [jnp.exp(vals[k] - vals[0]) for k in range(K)]
        s = es[0]
        for k in range(1, K):
            s = s + es[k]
        r = 1.0 / s
        for k in range(K):
            w_ref[rows, k:k + 1] = es[k] * r


@jax.jit
def kernel(hidden_states, wg_weight):
    wt = wg_weight.astype(jnp.float32).T
    x = hidden_states.astype(jnp.float32)
    grid = (T // BT,)
    idx, w = pl.pallas_call(
        _gate_kernel,
        grid=grid,
        in_specs=[
            pl.BlockSpec((BT, D), lambda i: (i, 0)),
            pl.BlockSpec((D, E), lambda i: (0, 0)),
        ],
        out_specs=[
            pl.BlockSpec((BT, K), lambda i: (i, 0)),
            pl.BlockSpec((BT, K), lambda i: (i, 0)),
        ],
        out_shape=[
            jax.ShapeDtypeStruct((T, K), jnp.int32),
            jax.ShapeDtypeStruct((T, K), jnp.float32),
        ],
    )(x, wt)
    return idx, w.astype(hidden_states.dtype)


# speedup vs baseline: 2.3006x; 1.3610x over previous
import jax
import jax.numpy as jnp
from jax.experimental import pallas as pl

T = 32768
D = 768
E = 64
K = 8
BT = 512
CHT = 512

NEG_INF = float("-inf")


def _gate_kernel(x_ref, wg_ref, idx_ref, w_ref):
    wg = wg_ref[...]
    subl = jax.lax.broadcasted_iota(jnp.int32, (E, CHT), 0).astype(jnp.float32)
    for c in range(BT // CHT):
        rows = pl.ds(c * CHT, CHT)
        x = x_ref[rows, :]
        lt = jax.lax.dot_general(
            wg, x, (((1,), (1,)), ((), ())),
            preferred_element_type=jnp.float32)

        vals = []
        work = lt
        for k in range(K):
            m = jnp.max(work, axis=0, keepdims=True)
            vals.append(m)
            if k + 1 < K:
                work = jnp.where(work == m, NEG_INF, work)

        idxs = []
        for k in range(K):
            hit = lt == vals[k]
            idxs.append(jnp.min(jnp.where(hit, subl, jnp.float32(E)),
                                axis=0, keepdims=True))

        vt = jnp.concatenate(vals, axis=0)
        it = jnp.concatenate(idxs, axis=0)
        e = jnp.exp(vt - vt[0:1, :])
        w = e * (1.0 / jnp.sum(e, axis=0, keepdims=True))
        idx_ref[rows, :] = it.T.astype(jnp.int32)
        w_ref[rows, :] = w.T


@jax.jit
def kernel(hidden_states, wg_weight):
    wg = wg_weight.astype(jnp.float32)
    x = hidden_states.astype(jnp.float32)
    grid = (T // BT,)
    idx, w = pl.pallas_call(
        _gate_kernel,
        grid=grid,
        in_specs=[
            pl.BlockSpec((BT, D), lambda i: (i, 0)),
            pl.BlockSpec((E, D), lambda i: (0, 0)),
        ],
        out_specs=[
            pl.BlockSpec((BT, K), lambda i: (i, 0)),
            pl.BlockSpec((BT, K), lambda i: (i, 0)),
        ],
        out_shape=[
            jax.ShapeDtypeStruct((T, K), jnp.int32),
            jax.ShapeDtypeStruct((T, K), jnp.float32),
        ],
    )(x, wg)
    return idx, w.astype(hidden_states.dtype)


# speedup vs baseline: 2.9586x; 1.2860x over previous
import jax
import jax.numpy as jnp
from jax.experimental import pallas as pl

T = 32768
D = 768
E = 64
K = 8
BT = 1024
CHT = 512

NEG_INF = float("-inf")


def _gate_kernel(x_ref, wg_ref, idx_ref, w_ref):
    wg = wg_ref[...]
    subl = jax.lax.broadcasted_iota(jnp.int32, (E, CHT), 0).astype(jnp.float32)
    for c in range(BT // CHT):
        rows = pl.ds(c * CHT, CHT)
        x = x_ref[rows, :]
        lt = jax.lax.dot_general(
            wg, x, (((1,), (1,)), ((), ())),
            preferred_element_type=jnp.float32)

        vals = []
        work = lt
        for k in range(K):
            m = jnp.max(work, axis=0, keepdims=True)
            vals.append(m)
            if k + 1 < K:
                work = jnp.where(work == m, NEG_INF, work)

        idxs = []
        for k in range(K):
            hit = lt == vals[k]
            idxs.append(jnp.min(jnp.where(hit, subl, jnp.float32(E)),
                                axis=0, keepdims=True))

        vt = jnp.concatenate(vals, axis=0)
        it = jnp.concatenate(idxs, axis=0)
        e = jnp.exp(vt - vt[0:1, :])
        w = e * (1.0 / jnp.sum(e, axis=0, keepdims=True))
        idx_ref[rows, :] = it.T.astype(jnp.int32)
        w_ref[rows, :] = w.T


@jax.jit
def kernel(hidden_states, wg_weight):
    wg = wg_weight.astype(jnp.float32)
    x = hidden_states.astype(jnp.float32)
    grid = (T // BT,)
    idx, w = pl.pallas_call(
        _gate_kernel,
        grid=grid,
        in_specs=[
            pl.BlockSpec((BT, D), lambda i: (i, 0)),
            pl.BlockSpec((E, D), lambda i: (0, 0)),
        ],
        out_specs=[
            pl.BlockSpec((BT, K), lambda i: (i, 0)),
            pl.BlockSpec((BT, K), lambda i: (i, 0)),
        ],
        out_shape=[
            jax.ShapeDtypeStruct((T, K), jnp.int32),
            jax.ShapeDtypeStruct((T, K), jnp.float32),
        ],
    )(x, wg)
    return idx, w.astype(hidden_states.dtype)


# speedup vs baseline: 3.3945x; 1.1473x over previous
import jax
import jax.numpy as jnp
from jax.experimental import pallas as pl

T = 32768
D = 768
E = 64
K = 8
BT = 2048
CHT = 512

NEG_INF = float("-inf")


def _gate_kernel(x_ref, wg_ref, idx_ref, w_ref):
    wg = wg_ref[...]
    subl = jax.lax.broadcasted_iota(jnp.int32, (E, CHT), 0).astype(jnp.float32)
    for c in range(BT // CHT):
        rows = pl.ds(c * CHT, CHT)
        x = x_ref[rows, :]
        lt = jax.lax.dot_general(
            wg, x, (((1,), (1,)), ((), ())),
            preferred_element_type=jnp.float32)

        vals = []
        work = lt
        for k in range(K):
            m = jnp.max(work, axis=0, keepdims=True)
            vals.append(m)
            if k + 1 < K:
                work = jnp.where(work == m, NEG_INF, work)

        idxs = []
        for k in range(K):
            hit = lt == vals[k]
            idxs.append(jnp.min(jnp.where(hit, subl, jnp.float32(E)),
                                axis=0, keepdims=True))

        vt = jnp.concatenate(vals, axis=0)
        it = jnp.concatenate(idxs, axis=0)
        e = jnp.exp(vt - vt[0:1, :])
        w = e * (1.0 / jnp.sum(e, axis=0, keepdims=True))
        idx_ref[rows, :] = it.T.astype(jnp.int32)
        w_ref[rows, :] = w.T


@jax.jit
def kernel(hidden_states, wg_weight):
    wg = wg_weight.astype(jnp.float32)
    x = hidden_states.astype(jnp.float32)
    grid = (T // BT,)
    idx, w = pl.pallas_call(
        _gate_kernel,
        grid=grid,
        in_specs=[
            pl.BlockSpec((BT, D), lambda i: (i, 0)),
            pl.BlockSpec((E, D), lambda i: (0, 0)),
        ],
        out_specs=[
            pl.BlockSpec((BT, K), lambda i: (i, 0)),
            pl.BlockSpec((BT, K), lambda i: (i, 0)),
        ],
        out_shape=[
            jax.ShapeDtypeStruct((T, K), jnp.int32),
            jax.ShapeDtypeStruct((T, K), jnp.float32),
        ],
    )(x, wg)
    return idx, w.astype(hidden_states.dtype)


# speedup vs baseline: 3.5192x; 1.0368x over previous
import jax
import jax.numpy as jnp
from jax.experimental import pallas as pl

T = 32768
D = 768
E = 64
K = 8
BT = 4096
CHT = 512

NEG_INF = float("-inf")


def _gate_kernel(x_ref, wg_ref, idx_ref, w_ref):
    wg = wg_ref[...]
    subl = jax.lax.broadcasted_iota(jnp.int32, (E, CHT), 0).astype(jnp.float32)
    for c in range(BT // CHT):
        rows = pl.ds(c * CHT, CHT)
        x = x_ref[rows, :]
        lt = jax.lax.dot_general(
            wg, x, (((1,), (1,)), ((), ())),
            preferred_element_type=jnp.float32)

        vals = []
        work = lt
        for k in range(K):
            m = jnp.max(work, axis=0, keepdims=True)
            vals.append(m)
            if k + 1 < K:
                work = jnp.where(work == m, NEG_INF, work)

        idxs = []
        for k in range(K):
            hit = lt == vals[k]
            idxs.append(jnp.min(jnp.where(hit, subl, jnp.float32(E)),
                                axis=0, keepdims=True))

        vt = jnp.concatenate(vals, axis=0)
        it = jnp.concatenate(idxs, axis=0)
        e = jnp.exp(vt - vt[0:1, :])
        w = e * (1.0 / jnp.sum(e, axis=0, keepdims=True))
        idx_ref[rows, :] = it.T.astype(jnp.int32)
        w_ref[rows, :] = w.T


@jax.jit
def kernel(hidden_states, wg_weight):
    wg = wg_weight.astype(jnp.float32)
    x = hidden_states.astype(jnp.float32)
    grid = (T // BT,)
    idx, w = pl.pallas_call(
        _gate_kernel,
        grid=grid,
        in_specs=[
            pl.BlockSpec((BT, D), lambda i: (i, 0)),
            pl.BlockSpec((E, D), lambda i: (0, 0)),
        ],
        out_specs=[
            pl.BlockSpec((BT, K), lambda i: (i, 0)),
            pl.BlockSpec((BT, K), lambda i: (i, 0)),
        ],
        out_shape=[
            jax.ShapeDtypeStruct((T, K), jnp.int32),
            jax.ShapeDtypeStruct((T, K), jnp.float32),
        ],
    )(x, wg)
    return idx, w.astype(hidden_states.dtype)
